# baseline (speedup 1.0000x reference)
import functools
import math

import jax
import jax.numpy as jnp
from jax import lax
from jax.experimental import pallas as pl
from jax.experimental.pallas import tpu as pltpu
from jax.experimental.pallas import tpu_sc as plsc

N = 10000
E = 160000
D = 256
H = 128
NS = 16
NC = 2
CH = 104
EPS = E // NS
EPSP = 10400
NCHUNK = EPSP // CH
HALF = NCHUNK // 2
NP = 10240
DUMP = NP - 1
ROWS_PT = NP // NS
EPW = E // (NS * NC)
DEG_ROWS = N // 16
BLK = 1000
GRID = N // BLK

_mesh = plsc.VectorSubcoreMesh(core_axis_name="c", subcore_axis_name="s")


@functools.partial(
    pl.kernel,
    out_type=jax.ShapeDtypeStruct((NC * NS, N), jnp.float32),
    mesh=_mesh,
    scratch_types=[
        pltpu.VMEM((EPW + 16,), jnp.int32),
        pltpu.VMEM((N,), jnp.float32),
    ],
    compiler_params=pltpu.CompilerParams(needs_layout_passes=False),
)
def _deg(dst_hbm, zeros_hbm, out_hbm, dstv, priv):
    c = lax.axis_index("c")
    s = lax.axis_index("s")
    wid = s * NC + c
    pltpu.sync_copy(dst_hbm.at[pl.ds(wid * EPW, EPW)], dstv.at[pl.ds(0, EPW)])
    pltpu.sync_copy(zeros_hbm, priv)

    iota = lax.iota(jnp.int32, 16)
    ones = jnp.full((16,), 1.0, jnp.float32)

    def body(q, carry):
        dv = dstv[pl.ds(q * 16, 16)]
        msk = (iota + q * 16) < EPW
        plsc.addupdate_scatter(priv, [dv], ones, mask=msk)
        return carry

    lax.fori_loop(0, (EPW + 15) // 16, body, 0)
    pltpu.sync_copy(priv, out_hbm.at[wid])


@functools.partial(
    pl.kernel,
    out_type=jax.ShapeDtypeStruct((NC, NP, H), jnp.float32),
    mesh=_mesh,
    scratch_types=[
        pltpu.VMEM((HALF, CH), jnp.int32),
        pltpu.VMEM((HALF, CH), jnp.int32),
        pltpu.VMEM((CH, H), jnp.float32),
        pltpu.VMEM((CH, H), jnp.float32),
        pltpu.VMEM_SHARED((NP, H), jnp.float32),
        pltpu.SemaphoreType.DMA,
        pltpu.SemaphoreType.DMA,
    ],
    compiler_params=pltpu.CompilerParams(needs_layout_passes=False),
)
def _agg(gp_hbm, srcm_hbm, dstm_hbm, zeros_hbm, out_hbm,
         srcv, dstv, rows0, rows1, acc, sem0, sem1):
    c = lax.axis_index("c")
    s = lax.axis_index("s")
    pltpu.sync_copy(zeros_hbm, acc.at[pl.ds(s * ROWS_PT, ROWS_PT)])
    plsc.subcore_barrier()

    gp = gp_hbm.at[c]

    def gather(r, buf, sem):
        pltpu.async_copy(gp.at[srcv.at[r]], buf, sem)

    def wait(r, buf, sem):
        pltpu.make_async_copy(gp.at[srcv.at[r]], buf, sem).wait()

    def scat(r, buf):
        pltpu.sync_copy(buf, acc.at[dstv.at[r]], add=True)

    for half in range(2):
        pltpu.sync_copy(srcm_hbm.at[s, half], srcv)
        pltpu.sync_copy(dstm_hbm.at[s, half], dstv)
        gather(0, rows0, sem0)

        def body(g, carry):
            r = 2 * g
            gather(r + 1, rows1, sem1)
            wait(r, rows0, sem0)
            scat(r, rows0)
            gather(r + 2, rows0, sem0)
            wait(r + 1, rows1, sem1)
            scat(r + 1, rows1)
            return carry

        lax.fori_loop(0, HALF // 2 - 1, body, 0)
        gather(HALF - 1, rows1, sem1)
        wait(HALF - 2, rows0, sem0)
        scat(HALF - 2, rows0)
        wait(HALF - 1, rows1, sem1)
        scat(HALF - 1, rows1)
    plsc.subcore_barrier()
    pltpu.sync_copy(acc.at[pl.ds(s * ROWS_PT, ROWS_PT)],
                    out_hbm.at[c, pl.ds(s * ROWS_PT, ROWS_PT)])


def _ln(h, g, b):
    mu = jnp.mean(h, axis=-1, keepdims=True)
    var = jnp.mean((h - mu) ** 2, axis=-1, keepdims=True)
    return (h - mu) * lax.rsqrt(var + 1e-5) * g + b


_DNUM = (((1,), (0,)), ((), ()))


def _mm(a, w_ref):
    return lax.dot_general(a.astype(jnp.bfloat16), w_ref[...], _DNUM,
                           preferred_element_type=jnp.float32)


def _stage_a_body(x_ref, deg2_ref, ln1g_ref, ln1b_ref, win_ref, bin_ref,
                  ln2g_ref, ln2b_ref, wg_ref, h_out, gp_out):
    x = x_ref[...]
    deg = jnp.sum(deg2_ref[...], axis=1, keepdims=True) + 1.0
    dinv = lax.rsqrt(deg)
    h = _ln(x, ln1g_ref[...], ln1b_ref[...])
    h = _mm(h, win_ref) + bin_ref[...]
    h = 0.5 * h * (1.0 + lax.erf(h * (1.0 / math.sqrt(2.0))))
    g = _ln(h, ln2g_ref[...], ln2b_ref[...])
    g = _mm(g, wg_ref)
    gp = dinv * g
    h_out[...] = h
    gp_out[0] = gp[:, :H]
    gp_out[1] = gp[:, H:]


def _stage_a(x, deg2, ln1g, ln1b, win, b_in, ln2g, ln2b, wg):
    return pl.pallas_call(
        _stage_a_body,
        grid=(GRID,),
        in_specs=[
            pl.BlockSpec((BLK, D), lambda i: (i, 0)),
            pl.BlockSpec((BLK, NC * NS), lambda i: (i, 0)),
            pl.BlockSpec((1, D), lambda i: (0, 0)),
            pl.BlockSpec((1, D), lambda i: (0, 0)),
            pl.BlockSpec((D, D), lambda i: (0, 0)),
            pl.BlockSpec((1, D), lambda i: (0, 0)),
            pl.BlockSpec((1, D), lambda i: (0, 0)),
            pl.BlockSpec((1, D), lambda i: (0, 0)),
            pl.BlockSpec((D, D), lambda i: (0, 0)),
        ],
        out_specs=[
            pl.BlockSpec((BLK, D), lambda i: (i, 0)),
            pl.BlockSpec((NC, BLK, H), lambda i: (0, i, 0)),
        ],
        out_shape=[
            jax.ShapeDtypeStruct((N, D), jnp.float32),
            jax.ShapeDtypeStruct((NC, N, H), jnp.float32),
        ],
    )(x, deg2, ln1g, ln1b, win, b_in, ln2g, ln2b, wg)


def _stage_b_body(x_ref, h_ref, gp_ref, agg_ref, deg2_ref, bg_ref, wout_ref,
                  bout_ref, out_ref):
    deg = jnp.sum(deg2_ref[...], axis=1, keepdims=True) + 1.0
    dinv = lax.rsqrt(deg)
    aggf = jnp.concatenate(
        [agg_ref[0] + gp_ref[0], agg_ref[1] + gp_ref[1]], axis=1)
    gate = jnp.tanh(dinv * aggf + bg_ref[...])
    m = gate * h_ref[...]
    out_ref[...] = x_ref[...] + _mm(m, wout_ref) + bout_ref[...]


def _stage_b(x, h, gp, agg, deg2, bg, wout, bout):
    return pl.pallas_call(
        _stage_b_body,
        grid=(GRID,),
        in_specs=[
            pl.BlockSpec((BLK, D), lambda i: (i, 0)),
            pl.BlockSpec((BLK, D), lambda i: (i, 0)),
            pl.BlockSpec((NC, BLK, H), lambda i: (0, i, 0)),
            pl.BlockSpec((NC, BLK, H), lambda i: (0, i, 0)),
            pl.BlockSpec((BLK, NC * NS), lambda i: (i, 0)),
            pl.BlockSpec((1, D), lambda i: (0, 0)),
            pl.BlockSpec((D, D), lambda i: (0, 0)),
            pl.BlockSpec((1, D), lambda i: (0, 0)),
        ],
        out_specs=pl.BlockSpec((BLK, D), lambda i: (i, 0)),
        out_shape=jax.ShapeDtypeStruct((N, D), jnp.float32),
    )(x, h, gp, agg, deg2, bg, wout, bout)


@jax.jit
def kernel(x, edge_index, ln1_g, ln1_b, Win, b_in, ln2_g, ln2_b, Wg, bg,
           Wout, bout):
    src = edge_index[0].astype(jnp.int32)
    dst = edge_index[1].astype(jnp.int32)
    npad = EPSP - EPS
    srcm = jnp.zeros((NS, EPSP), jnp.int32)
    srcm = srcm.at[:, :EPS].set(src.reshape(NS, EPS))
    srcm = srcm.reshape(NS, 2, HALF, CH)
    k = jnp.arange(npad, dtype=jnp.int32)
    padrows = (N + 15 * jnp.arange(NS, dtype=jnp.int32)[:, None]
               + (k[None, :] % 15))
    dstm = jnp.concatenate([dst.reshape(NS, EPS), padrows], axis=1)
    dstm = dstm.reshape(NS, 2, HALF, CH)
    zeros_deg = jnp.zeros((N,), jnp.float32)
    zeros_agg = jnp.zeros((ROWS_PT, H), jnp.float32)

    WinT = jnp.swapaxes(Win, 1, 2).astype(jnp.bfloat16)
    WgT = jnp.swapaxes(Wg, 1, 2).astype(jnp.bfloat16)
    WoutT = jnp.swapaxes(Wout, 1, 2).astype(jnp.bfloat16)

    deg2 = _deg(dst, zeros_deg).T

    for i in range(3):
        h, gp = _stage_a(x, deg2, ln1_g[i:i + 1], ln1_b[i:i + 1], WinT[i],
                         b_in[i:i + 1], ln2_g[i:i + 1], ln2_b[i:i + 1],
                         WgT[i])
        agg = _agg(gp, srcm, dstm, zeros_agg)
        x = _stage_b(x, h, gp, agg, deg2, bg[i:i + 1], WoutT[i],
                     bout[i:i + 1])
    return x


# speedup vs baseline: 2.4873x; 2.4873x over previous
import functools
import math

import jax
import jax.numpy as jnp
from jax import lax
from jax.experimental import pallas as pl
from jax.experimental.pallas import tpu as pltpu
from jax.experimental.pallas import tpu_sc as plsc

N = 10000
E = 160000
D = 256
H = 128
NS = 16
NC = 2
CH = 120
EPS = E // NS
EPSP = 10080
NCHUNK = EPSP // CH
HALF = NCHUNK // 2
NP = 10240
DUMP = NP - 1
ROWS_PT = NP // NS
EPW = E // (NS * NC)
DEG_ROWS = N // 16
BLK = 1000
GRID = N // BLK

_mesh = plsc.VectorSubcoreMesh(core_axis_name="c", subcore_axis_name="s")


@functools.partial(
    pl.kernel,
    out_type=jax.ShapeDtypeStruct((NC * NS, N), jnp.float32),
    mesh=_mesh,
    scratch_types=[
        pltpu.VMEM((EPW + 16,), jnp.int32),
        pltpu.VMEM((N,), jnp.float32),
    ],
    compiler_params=pltpu.CompilerParams(needs_layout_passes=False),
)
def _deg(dst_hbm, zeros_hbm, out_hbm, dstv, priv):
    c = lax.axis_index("c")
    s = lax.axis_index("s")
    wid = s * NC + c
    pltpu.sync_copy(dst_hbm.at[pl.ds(wid * EPW, EPW)], dstv.at[pl.ds(0, EPW)])
    pltpu.sync_copy(zeros_hbm, priv)

    iota = lax.iota(jnp.int32, 16)
    ones = jnp.full((16,), 1.0, jnp.float32)

    def body(q, carry):
        dv = dstv[pl.ds(q * 16, 16)]
        msk = (iota + q * 16) < EPW
        plsc.addupdate_scatter(priv, [dv], ones, mask=msk)
        return carry

    lax.fori_loop(0, (EPW + 15) // 16, body, 0)
    pltpu.sync_copy(priv, out_hbm.at[wid])


@functools.partial(
    pl.kernel,
    out_type=jax.ShapeDtypeStruct((NC, NP, H), jnp.float32),
    mesh=_mesh,
    scratch_types=[
        pltpu.VMEM((HALF, CH), jnp.int32),
        pltpu.VMEM((HALF, CH), jnp.int32),
        pltpu.VMEM((CH, H), jnp.float32),
        pltpu.VMEM((CH, H), jnp.float32),
        pltpu.VMEM_SHARED((NP, H), jnp.float32),
        pltpu.SemaphoreType.DMA,
        pltpu.SemaphoreType.DMA,
    ],
    compiler_params=pltpu.CompilerParams(needs_layout_passes=False),
)
def _agg(gp_hbm, srcm_hbm, dstm_hbm, zeros_hbm, out_hbm,
         srcv, dstv, rows0, rows1, acc, sem0, sem1):
    c = lax.axis_index("c")
    s = lax.axis_index("s")
    pltpu.sync_copy(zeros_hbm, acc.at[pl.ds(s * ROWS_PT, ROWS_PT)])
    plsc.subcore_barrier()

    gp = gp_hbm.at[c]

    def gather(r, buf, sem):
        pltpu.async_copy(gp.at[srcv.at[r]], buf, sem)

    def wait(r, buf, sem):
        pltpu.make_async_copy(gp.at[srcv.at[r]], buf, sem).wait()

    def scat(r, buf):
        pltpu.sync_copy(buf, acc.at[dstv.at[r]], add=True)

    for half in range(2):
        pltpu.sync_copy(srcm_hbm.at[s, half], srcv)
        pltpu.sync_copy(dstm_hbm.at[s, half], dstv)
        gather(0, rows0, sem0)

        def body(g, carry):
            r = 2 * g
            gather(r + 1, rows1, sem1)
            wait(r, rows0, sem0)
            scat(r, rows0)
            gather(r + 2, rows0, sem0)
            wait(r + 1, rows1, sem1)
            scat(r + 1, rows1)
            return carry

        lax.fori_loop(0, HALF // 2 - 1, body, 0)
        gather(HALF - 1, rows1, sem1)
        wait(HALF - 2, rows0, sem0)
        scat(HALF - 2, rows0)
        wait(HALF - 1, rows1, sem1)
        scat(HALF - 1, rows1)
    plsc.subcore_barrier()
    pltpu.sync_copy(acc.at[pl.ds(s * ROWS_PT, ROWS_PT)],
                    out_hbm.at[c, pl.ds(s * ROWS_PT, ROWS_PT)])


def _ln(h, g, b):
    mu = jnp.mean(h, axis=-1, keepdims=True)
    var = jnp.mean((h - mu) ** 2, axis=-1, keepdims=True)
    return (h - mu) * lax.rsqrt(var + 1e-5) * g + b


_DNUM = (((1,), (0,)), ((), ()))


def _mm(a, w_ref):
    return lax.dot_general(a.astype(jnp.bfloat16), w_ref[...], _DNUM,
                           preferred_element_type=jnp.float32)


def _stage_a_body(x_ref, deg2_ref, ln1g_ref, ln1b_ref, win_ref, bin_ref,
                  ln2g_ref, ln2b_ref, wg_ref, h_out, gp_out):
    x = x_ref[...]
    deg = jnp.sum(deg2_ref[...], axis=1, keepdims=True) + 1.0
    dinv = lax.rsqrt(deg)
    h = _ln(x, ln1g_ref[...], ln1b_ref[...])
    h = _mm(h, win_ref) + bin_ref[...]
    h = 0.5 * h * (1.0 + lax.erf(h * (1.0 / math.sqrt(2.0))))
    g = _ln(h, ln2g_ref[...], ln2b_ref[...])
    g = _mm(g, wg_ref)
    gp = dinv * g
    h_out[...] = h
    gp_out[0] = gp[:, :H]
    gp_out[1] = gp[:, H:]


def _stage_a(x, deg2, ln1g, ln1b, win, b_in, ln2g, ln2b, wg):
    return pl.pallas_call(
        _stage_a_body,
        grid=(GRID,),
        in_specs=[
            pl.BlockSpec((BLK, D), lambda i: (i, 0)),
            pl.BlockSpec((BLK, NC * NS), lambda i: (i, 0)),
            pl.BlockSpec((1, D), lambda i: (0, 0)),
            pl.BlockSpec((1, D), lambda i: (0, 0)),
            pl.BlockSpec((D, D), lambda i: (0, 0)),
            pl.BlockSpec((1, D), lambda i: (0, 0)),
            pl.BlockSpec((1, D), lambda i: (0, 0)),
            pl.BlockSpec((1, D), lambda i: (0, 0)),
            pl.BlockSpec((D, D), lambda i: (0, 0)),
        ],
        out_specs=[
            pl.BlockSpec((BLK, D), lambda i: (i, 0)),
            pl.BlockSpec((NC, BLK, H), lambda i: (0, i, 0)),
        ],
        out_shape=[
            jax.ShapeDtypeStruct((N, D), jnp.float32),
            jax.ShapeDtypeStruct((NC, N, H), jnp.float32),
        ],
    )(x, deg2, ln1g, ln1b, win, b_in, ln2g, ln2b, wg)


def _stage_b_body(x_ref, h_ref, gp_ref, agg_ref, deg2_ref, bg_ref, wout_ref,
                  bout_ref, out_ref):
    deg = jnp.sum(deg2_ref[...], axis=1, keepdims=True) + 1.0
    dinv = lax.rsqrt(deg)
    aggf = jnp.concatenate(
        [agg_ref[0] + gp_ref[0], agg_ref[1] + gp_ref[1]], axis=1)
    gate = jnp.tanh(dinv * aggf + bg_ref[...])
    m = gate * h_ref[...]
    out_ref[...] = x_ref[...] + _mm(m, wout_ref) + bout_ref[...]


def _stage_b(x, h, gp, agg, deg2, bg, wout, bout):
    return pl.pallas_call(
        _stage_b_body,
        grid=(GRID,),
        in_specs=[
            pl.BlockSpec((BLK, D), lambda i: (i, 0)),
            pl.BlockSpec((BLK, D), lambda i: (i, 0)),
            pl.BlockSpec((NC, BLK, H), lambda i: (0, i, 0)),
            pl.BlockSpec((NC, BLK, H), lambda i: (0, i, 0)),
            pl.BlockSpec((BLK, NC * NS), lambda i: (i, 0)),
            pl.BlockSpec((1, D), lambda i: (0, 0)),
            pl.BlockSpec((D, D), lambda i: (0, 0)),
            pl.BlockSpec((1, D), lambda i: (0, 0)),
        ],
        out_specs=pl.BlockSpec((BLK, D), lambda i: (i, 0)),
        out_shape=jax.ShapeDtypeStruct((N, D), jnp.float32),
    )(x, h, gp, agg, deg2, bg, wout, bout)


@jax.jit
def kernel(x, edge_index, ln1_g, ln1_b, Win, b_in, ln2_g, ln2_b, Wg, bg,
           Wout, bout):
    src = edge_index[0].astype(jnp.int32)
    dst = edge_index[1].astype(jnp.int32)
    npad = EPSP - EPS
    k = jnp.arange(npad, dtype=jnp.int32)
    padsrc = jnp.broadcast_to((k * 37) % N, (NS, npad))
    srcm = jnp.concatenate([src.reshape(NS, EPS), padsrc], axis=1)
    srcm = srcm.reshape(NS, 2, HALF, CH)
    padrows = (N + 15 * jnp.arange(NS, dtype=jnp.int32)[:, None]
               + (k[None, :] % 15))
    dstm = jnp.concatenate([dst.reshape(NS, EPS), padrows], axis=1)
    dstm = dstm.reshape(NS, 2, HALF, CH)
    zeros_deg = jnp.zeros((N,), jnp.float32)
    zeros_agg = jnp.zeros((ROWS_PT, H), jnp.float32)

    WinT = jnp.swapaxes(Win, 1, 2).astype(jnp.bfloat16)
    WgT = jnp.swapaxes(Wg, 1, 2).astype(jnp.bfloat16)
    WoutT = jnp.swapaxes(Wout, 1, 2).astype(jnp.bfloat16)

    deg2 = _deg(dst, zeros_deg).T

    for i in range(3):
        h, gp = _stage_a(x, deg2, ln1_g[i:i + 1], ln1_b[i:i + 1], WinT[i],
                         b_in[i:i + 1], ln2_g[i:i + 1], ln2_b[i:i + 1],
                         WgT[i])
        agg = _agg(gp, srcm, dstm, zeros_agg)
        x = _stage_b(x, h, gp, agg, deg2, bg[i:i + 1], WoutT[i],
                     bout[i:i + 1])
    return x


# speedup vs baseline: 2.5013x; 1.0056x over previous
import functools
import math

import jax
import jax.numpy as jnp
from jax import lax
from jax.experimental import pallas as pl
from jax.experimental.pallas import tpu as pltpu
from jax.experimental.pallas import tpu_sc as plsc

N = 10000
E = 160000
D = 256
H = 128
NS = 16
NC = 2
CH = 120
EPS = E // NS
EPSP = 10080
NCHUNK = EPSP // CH
HALF = NCHUNK // 2
NP = 10240
DUMP = NP - 1
ROWS_PT = NP // NS
EPW = E // (NS * NC)
DEG_ROWS = N // 16
BLK = 1000
GRID = N // BLK

_mesh = plsc.VectorSubcoreMesh(core_axis_name="c", subcore_axis_name="s")


@functools.partial(
    pl.kernel,
    out_type=jax.ShapeDtypeStruct((NC * NS, N), jnp.float32),
    mesh=_mesh,
    scratch_types=[
        pltpu.VMEM((EPW + 16,), jnp.int32),
        pltpu.VMEM((N,), jnp.float32),
    ],
    compiler_params=pltpu.CompilerParams(needs_layout_passes=False),
)
def _deg(dst_hbm, zeros_hbm, out_hbm, dstv, priv):
    c = lax.axis_index("c")
    s = lax.axis_index("s")
    wid = s * NC + c
    pltpu.sync_copy(dst_hbm.at[pl.ds(wid * EPW, EPW)], dstv.at[pl.ds(0, EPW)])
    pltpu.sync_copy(zeros_hbm, priv)

    iota = lax.iota(jnp.int32, 16)
    ones = jnp.full((16,), 1.0, jnp.float32)

    def body(q, carry):
        dv = dstv[pl.ds(q * 16, 16)]
        msk = (iota + q * 16) < EPW
        plsc.addupdate_scatter(priv, [dv], ones, mask=msk)
        return carry

    lax.fori_loop(0, (EPW + 15) // 16, body, 0)
    pltpu.sync_copy(priv, out_hbm.at[wid])


@functools.partial(
    pl.kernel,
    out_type=jax.ShapeDtypeStruct((NC, NP, H), jnp.float32),
    mesh=_mesh,
    scratch_types=[
        pltpu.VMEM((HALF, CH), jnp.int32),
        pltpu.VMEM((HALF, CH), jnp.int32),
        pltpu.VMEM((CH, H), jnp.float32),
        pltpu.VMEM((CH, H), jnp.float32),
        pltpu.VMEM_SHARED((NP, H), jnp.float32),
        pltpu.SemaphoreType.DMA,
        pltpu.SemaphoreType.DMA,
    ],
    compiler_params=pltpu.CompilerParams(needs_layout_passes=False),
)
def _agg(gp_hbm, srcm_hbm, dstm_hbm, zeros_hbm, out_hbm,
         srcv, dstv, rows0, rows1, acc, sem0, sem1):
    c = lax.axis_index("c")
    s = lax.axis_index("s")

    gp = gp_hbm.at[c]

    def gather(r, buf, sem):
        pltpu.async_copy(gp.at[srcv.at[r]], buf, sem)

    def wait(r, buf, sem):
        pltpu.make_async_copy(gp.at[srcv.at[r]], buf, sem).wait()

    def scat(r, buf):
        pltpu.sync_copy(buf, acc.at[dstv.at[r]], add=True)

    for half in range(2):
        pltpu.sync_copy(srcm_hbm.at[s, half], srcv)
        pltpu.sync_copy(dstm_hbm.at[s, half], dstv)
        gather(0, rows0, sem0)
        if half == 0:
            pltpu.sync_copy(zeros_hbm, acc.at[pl.ds(s * ROWS_PT, ROWS_PT)])
            plsc.subcore_barrier()

        def body(g, carry):
            r = 2 * g
            gather(r + 1, rows1, sem1)
            wait(r, rows0, sem0)
            scat(r, rows0)
            gather(r + 2, rows0, sem0)
            wait(r + 1, rows1, sem1)
            scat(r + 1, rows1)
            return carry

        lax.fori_loop(0, HALF // 2 - 1, body, 0)
        gather(HALF - 1, rows1, sem1)
        wait(HALF - 2, rows0, sem0)
        scat(HALF - 2, rows0)
        wait(HALF - 1, rows1, sem1)
        scat(HALF - 1, rows1)
    plsc.subcore_barrier()
    pltpu.sync_copy(acc.at[pl.ds(s * ROWS_PT, ROWS_PT)],
                    out_hbm.at[c, pl.ds(s * ROWS_PT, ROWS_PT)])


def _ln(h, g, b):
    mu = jnp.mean(h, axis=-1, keepdims=True)
    var = jnp.mean((h - mu) ** 2, axis=-1, keepdims=True)
    return (h - mu) * lax.rsqrt(var + 1e-5) * g + b


_DNUM = (((1,), (0,)), ((), ()))


def _mm(a, w_ref):
    return lax.dot_general(a.astype(jnp.bfloat16), w_ref[...], _DNUM,
                           preferred_element_type=jnp.float32)


def _stage_a_body(x_ref, deg2_ref, ln1g_ref, ln1b_ref, win_ref, bin_ref,
                  ln2g_ref, ln2b_ref, wg_ref, h_out, gp_out):
    x = x_ref[...]
    deg = jnp.sum(deg2_ref[...], axis=1, keepdims=True) + 1.0
    dinv = lax.rsqrt(deg)
    h = _ln(x, ln1g_ref[...], ln1b_ref[...])
    h = _mm(h, win_ref) + bin_ref[...]
    h = 0.5 * h * (1.0 + lax.erf(h * (1.0 / math.sqrt(2.0))))
    g = _ln(h, ln2g_ref[...], ln2b_ref[...])
    g = _mm(g, wg_ref)
    gp = dinv * g
    h_out[...] = h
    gp_out[0] = gp[:, :H]
    gp_out[1] = gp[:, H:]


def _stage_a(x, deg2, ln1g, ln1b, win, b_in, ln2g, ln2b, wg):
    return pl.pallas_call(
        _stage_a_body,
        grid=(GRID,),
        in_specs=[
            pl.BlockSpec((BLK, D), lambda i: (i, 0)),
            pl.BlockSpec((BLK, NC * NS), lambda i: (i, 0)),
            pl.BlockSpec((1, D), lambda i: (0, 0)),
            pl.BlockSpec((1, D), lambda i: (0, 0)),
            pl.BlockSpec((D, D), lambda i: (0, 0)),
            pl.BlockSpec((1, D), lambda i: (0, 0)),
            pl.BlockSpec((1, D), lambda i: (0, 0)),
            pl.BlockSpec((1, D), lambda i: (0, 0)),
            pl.BlockSpec((D, D), lambda i: (0, 0)),
        ],
        out_specs=[
            pl.BlockSpec((BLK, D), lambda i: (i, 0)),
            pl.BlockSpec((NC, BLK, H), lambda i: (0, i, 0)),
        ],
        out_shape=[
            jax.ShapeDtypeStruct((N, D), jnp.float32),
            jax.ShapeDtypeStruct((NC, N, H), jnp.float32),
        ],
    )(x, deg2, ln1g, ln1b, win, b_in, ln2g, ln2b, wg)


def _stage_b_body(x_ref, h_ref, gp_ref, agg_ref, deg2_ref, bg_ref, wout_ref,
                  bout_ref, out_ref):
    deg = jnp.sum(deg2_ref[...], axis=1, keepdims=True) + 1.0
    dinv = lax.rsqrt(deg)
    aggf = jnp.concatenate(
        [agg_ref[0] + gp_ref[0], agg_ref[1] + gp_ref[1]], axis=1)
    gate = jnp.tanh(dinv * aggf + bg_ref[...])
    m = gate * h_ref[...]
    out_ref[...] = x_ref[...] + _mm(m, wout_ref) + bout_ref[...]


def _stage_b(x, h, gp, agg, deg2, bg, wout, bout):
    return pl.pallas_call(
        _stage_b_body,
        grid=(GRID,),
        in_specs=[
            pl.BlockSpec((BLK, D), lambda i: (i, 0)),
            pl.BlockSpec((BLK, D), lambda i: (i, 0)),
            pl.BlockSpec((NC, BLK, H), lambda i: (0, i, 0)),
            pl.BlockSpec((NC, BLK, H), lambda i: (0, i, 0)),
            pl.BlockSpec((BLK, NC * NS), lambda i: (i, 0)),
            pl.BlockSpec((1, D), lambda i: (0, 0)),
            pl.BlockSpec((D, D), lambda i: (0, 0)),
            pl.BlockSpec((1, D), lambda i: (0, 0)),
        ],
        out_specs=pl.BlockSpec((BLK, D), lambda i: (i, 0)),
        out_shape=jax.ShapeDtypeStruct((N, D), jnp.float32),
    )(x, h, gp, agg, deg2, bg, wout, bout)


@jax.jit
def kernel(x, edge_index, ln1_g, ln1_b, Win, b_in, ln2_g, ln2_b, Wg, bg,
           Wout, bout):
    src = edge_index[0].astype(jnp.int32)
    dst = edge_index[1].astype(jnp.int32)
    npad = EPSP - EPS
    k = jnp.arange(npad, dtype=jnp.int32)
    padsrc = jnp.broadcast_to((k * 37) % N, (NS, npad))
    srcm = jnp.concatenate([src.reshape(NS, EPS), padsrc], axis=1)
    srcm = srcm.reshape(NS, 2, HALF, CH)
    padrows = (N + 15 * jnp.arange(NS, dtype=jnp.int32)[:, None]
               + (k[None, :] % 15))
    dstm = jnp.concatenate([dst.reshape(NS, EPS), padrows], axis=1)
    dstm = dstm.reshape(NS, 2, HALF, CH)
    zeros_deg = jnp.zeros((N,), jnp.float32)
    zeros_agg = jnp.zeros((ROWS_PT, H), jnp.float32)

    WinT = jnp.swapaxes(Win, 1, 2).astype(jnp.bfloat16)
    WgT = jnp.swapaxes(Wg, 1, 2).astype(jnp.bfloat16)
    WoutT = jnp.swapaxes(Wout, 1, 2).astype(jnp.bfloat16)

    deg2 = _deg(dst, zeros_deg).T

    for i in range(3):
        h, gp = _stage_a(x, deg2, ln1_g[i:i + 1], ln1_b[i:i + 1], WinT[i],
                         b_in[i:i + 1], ln2_g[i:i + 1], ln2_b[i:i + 1],
                         WgT[i])
        agg = _agg(gp, srcm, dstm, zeros_agg)
        x = _stage_b(x, h, gp, agg, deg2, bg[i:i + 1], WoutT[i],
                     bout[i:i + 1])
    return x


# speedup vs baseline: 2.6149x; 1.0454x over previous
import functools
import math

import jax
import jax.numpy as jnp
from jax import lax
from jax.experimental import pallas as pl
from jax.experimental.pallas import tpu as pltpu
from jax.experimental.pallas import tpu_sc as plsc

N = 10000
E = 160000
D = 256
H = 128
NS = 16
NC = 2
CH = 120
EPS = E // NS
EPSP = 10080
NCHUNK = EPSP // CH
HALF = NCHUNK // 2
NP = 10240
DUMP = NP - 1
ROWS_PT = NP // NS
EPW = E // (NS * NC)
DEG_ROWS = N // 16
BLK = 1000
GRID = N // BLK

_mesh = plsc.VectorSubcoreMesh(core_axis_name="c", subcore_axis_name="s")


@functools.partial(
    pl.kernel,
    out_type=jax.ShapeDtypeStruct((NC * NS, N), jnp.float32),
    mesh=_mesh,
    scratch_types=[
        pltpu.VMEM((EPW + 16,), jnp.int32),
        pltpu.VMEM((N,), jnp.float32),
    ],
    compiler_params=pltpu.CompilerParams(needs_layout_passes=False),
)
def _deg(dst_hbm, zeros_hbm, out_hbm, dstv, priv):
    c = lax.axis_index("c")
    s = lax.axis_index("s")
    wid = s * NC + c
    pltpu.sync_copy(dst_hbm.at[pl.ds(wid * EPW, EPW)], dstv.at[pl.ds(0, EPW)])
    pltpu.sync_copy(zeros_hbm, priv)

    iota = lax.iota(jnp.int32, 16)
    ones = jnp.full((16,), 1.0, jnp.float32)

    def body(q, carry):
        dv = dstv[pl.ds(q * 16, 16)]
        msk = (iota + q * 16) < EPW
        plsc.addupdate_scatter(priv, [dv], ones, mask=msk)
        return carry

    lax.fori_loop(0, (EPW + 15) // 16, body, 0)
    pltpu.sync_copy(priv, out_hbm.at[wid])


@functools.partial(
    pl.kernel,
    out_type=jax.ShapeDtypeStruct((NC, NP, H), jnp.float32),
    mesh=_mesh,
    scratch_types=[
        pltpu.VMEM((HALF, CH), jnp.int32),
        pltpu.VMEM((HALF, CH), jnp.int32),
        pltpu.VMEM((CH, H), jnp.float32),
        pltpu.VMEM((CH, H), jnp.float32),
        pltpu.VMEM_SHARED((NP, H), jnp.float32),
        pltpu.SemaphoreType.DMA,
        pltpu.SemaphoreType.DMA,
    ],
    compiler_params=pltpu.CompilerParams(needs_layout_passes=False),
)
def _agg(gp_hbm, srcm_hbm, dstm_hbm, zeros_hbm, out_hbm,
         srcv, dstv, rows0, rows1, acc, sem0, sem1):
    c = lax.axis_index("c")
    s = lax.axis_index("s")

    gp = gp_hbm.at[c]

    def gather(r, buf, sem):
        pltpu.async_copy(gp.at[srcv.at[r]], buf, sem)

    def wait(r, buf, sem):
        pltpu.make_async_copy(gp.at[srcv.at[r]], buf, sem).wait()

    def scat(r, buf):
        pltpu.sync_copy(buf, acc.at[dstv.at[r]], add=True)

    for half in range(2):
        pltpu.sync_copy(srcm_hbm.at[s, half], srcv)
        pltpu.sync_copy(dstm_hbm.at[s, half], dstv)
        gather(0, rows0, sem0)
        if half == 0:
            pltpu.sync_copy(zeros_hbm, acc.at[pl.ds(s * ROWS_PT, ROWS_PT)])
            plsc.subcore_barrier()

        def body(g, carry):
            r = 2 * g
            gather(r + 1, rows1, sem1)
            wait(r, rows0, sem0)
            scat(r, rows0)
            gather(r + 2, rows0, sem0)
            wait(r + 1, rows1, sem1)
            scat(r + 1, rows1)
            return carry

        lax.fori_loop(0, HALF // 2 - 1, body, 0)
        gather(HALF - 1, rows1, sem1)
        wait(HALF - 2, rows0, sem0)
        scat(HALF - 2, rows0)
        wait(HALF - 1, rows1, sem1)
        scat(HALF - 1, rows1)
    plsc.subcore_barrier()
    pltpu.sync_copy(acc.at[pl.ds(s * ROWS_PT, ROWS_PT)],
                    out_hbm.at[c, pl.ds(s * ROWS_PT, ROWS_PT)])


def _ln(h, g, b):
    mu = jnp.mean(h, axis=-1, keepdims=True)
    var = jnp.mean((h - mu) ** 2, axis=-1, keepdims=True)
    return (h - mu) * lax.rsqrt(var + 1e-5) * g + b


_DNUM = (((1,), (0,)), ((), ()))


def _mm(a, w_ref):
    return lax.dot_general(a.astype(jnp.bfloat16), w_ref[...], _DNUM,
                           preferred_element_type=jnp.float32)


def _stage_a_body(x_ref, deg2_ref, ln1g_ref, ln1b_ref, win_ref, bin_ref,
                  ln2g_ref, ln2b_ref, wg_ref, h_out, gp_out):
    x = x_ref[...]
    deg = jnp.sum(deg2_ref[...], axis=1, keepdims=True) + 1.0
    dinv = lax.rsqrt(deg)
    h = _ln(x, ln1g_ref[...], ln1b_ref[...])
    h = _mm(h, win_ref) + bin_ref[...]
    h = 0.5 * h * (1.0 + lax.erf(h * (1.0 / math.sqrt(2.0))))
    g = _ln(h, ln2g_ref[...], ln2b_ref[...])
    g = _mm(g, wg_ref)
    gp = dinv * g
    h_out[...] = h
    gp_out[0] = gp[:, :H]
    gp_out[1] = gp[:, H:]


def _stage_a(x, deg2, ln1g, ln1b, win, b_in, ln2g, ln2b, wg):
    return pl.pallas_call(
        _stage_a_body,
        grid=(GRID,),
        in_specs=[
            pl.BlockSpec((BLK, D), lambda i: (i, 0)),
            pl.BlockSpec((BLK, NC * NS), lambda i: (i, 0)),
            pl.BlockSpec((1, D), lambda i: (0, 0)),
            pl.BlockSpec((1, D), lambda i: (0, 0)),
            pl.BlockSpec((D, D), lambda i: (0, 0)),
            pl.BlockSpec((1, D), lambda i: (0, 0)),
            pl.BlockSpec((1, D), lambda i: (0, 0)),
            pl.BlockSpec((1, D), lambda i: (0, 0)),
            pl.BlockSpec((D, D), lambda i: (0, 0)),
        ],
        out_specs=[
            pl.BlockSpec((BLK, D), lambda i: (i, 0)),
            pl.BlockSpec((NC, BLK, H), lambda i: (0, i, 0)),
        ],
        out_shape=[
            jax.ShapeDtypeStruct((N, D), jnp.float32),
            jax.ShapeDtypeStruct((NC, N, H), jnp.float32),
        ],
    )(x, deg2, ln1g, ln1b, win, b_in, ln2g, ln2b, wg)


def _stage_b_body(x_ref, h_ref, gp_ref, agg_ref, deg2_ref, bg_ref, wout_ref,
                  bout_ref, out_ref):
    deg = jnp.sum(deg2_ref[...], axis=1, keepdims=True) + 1.0
    dinv = lax.rsqrt(deg)
    aggf = jnp.concatenate(
        [agg_ref[0] + gp_ref[0], agg_ref[1] + gp_ref[1]], axis=1)
    gate = jnp.tanh(dinv * aggf + bg_ref[...])
    m = gate * h_ref[...]
    out_ref[...] = x_ref[...] + _mm(m, wout_ref) + bout_ref[...]


def _stage_b(x, h, gp, agg, deg2, bg, wout, bout):
    return pl.pallas_call(
        _stage_b_body,
        grid=(GRID,),
        in_specs=[
            pl.BlockSpec((BLK, D), lambda i: (i, 0)),
            pl.BlockSpec((BLK, D), lambda i: (i, 0)),
            pl.BlockSpec((NC, BLK, H), lambda i: (0, i, 0)),
            pl.BlockSpec((NC, BLK, H), lambda i: (0, i, 0)),
            pl.BlockSpec((BLK, NC * NS), lambda i: (i, 0)),
            pl.BlockSpec((1, D), lambda i: (0, 0)),
            pl.BlockSpec((D, D), lambda i: (0, 0)),
            pl.BlockSpec((1, D), lambda i: (0, 0)),
        ],
        out_specs=pl.BlockSpec((BLK, D), lambda i: (i, 0)),
        out_shape=jax.ShapeDtypeStruct((N, D), jnp.float32),
    )(x, h, gp, agg, deg2, bg, wout, bout)


def _stage_ba_body(x_ref, h_ref, gp_ref, agg_ref, deg2_ref, bg_ref, wout_ref,
                   bout_ref, ln1g_ref, ln1b_ref, win_ref, bin_ref, ln2g_ref,
                   ln2b_ref, wg_ref, x_out, h_out, gp_out):
    deg = jnp.sum(deg2_ref[...], axis=1, keepdims=True) + 1.0
    dinv = lax.rsqrt(deg)
    aggf = jnp.concatenate(
        [agg_ref[0] + gp_ref[0], agg_ref[1] + gp_ref[1]], axis=1)
    gate = jnp.tanh(dinv * aggf + bg_ref[...])
    m = gate * h_ref[...]
    xn = x_ref[...] + _mm(m, wout_ref) + bout_ref[...]
    x_out[...] = xn
    h = _ln(xn, ln1g_ref[...], ln1b_ref[...])
    h = _mm(h, win_ref) + bin_ref[...]
    h = 0.5 * h * (1.0 + lax.erf(h * (1.0 / math.sqrt(2.0))))
    g = _ln(h, ln2g_ref[...], ln2b_ref[...])
    g = _mm(g, wg_ref)
    gp = dinv * g
    h_out[...] = h
    gp_out[0] = gp[:, :H]
    gp_out[1] = gp[:, H:]


def _stage_ba(x, h, gp, agg, deg2, bg, wout, bout,
              ln1g, ln1b, win, b_in, ln2g, ln2b, wg):
    full = lambda i: (0, 0)
    return pl.pallas_call(
        _stage_ba_body,
        grid=(GRID,),
        in_specs=[
            pl.BlockSpec((BLK, D), lambda i: (i, 0)),
            pl.BlockSpec((BLK, D), lambda i: (i, 0)),
            pl.BlockSpec((NC, BLK, H), lambda i: (0, i, 0)),
            pl.BlockSpec((NC, BLK, H), lambda i: (0, i, 0)),
            pl.BlockSpec((BLK, NC * NS), lambda i: (i, 0)),
            pl.BlockSpec((1, D), full),
            pl.BlockSpec((D, D), full),
            pl.BlockSpec((1, D), full),
            pl.BlockSpec((1, D), full),
            pl.BlockSpec((1, D), full),
            pl.BlockSpec((D, D), full),
            pl.BlockSpec((1, D), full),
            pl.BlockSpec((1, D), full),
            pl.BlockSpec((1, D), full),
            pl.BlockSpec((D, D), full),
        ],
        out_specs=[
            pl.BlockSpec((BLK, D), lambda i: (i, 0)),
            pl.BlockSpec((BLK, D), lambda i: (i, 0)),
            pl.BlockSpec((NC, BLK, H), lambda i: (0, i, 0)),
        ],
        out_shape=[
            jax.ShapeDtypeStruct((N, D), jnp.float32),
            jax.ShapeDtypeStruct((N, D), jnp.float32),
            jax.ShapeDtypeStruct((NC, N, H), jnp.float32),
        ],
    )(x, h, gp, agg, deg2, bg, wout, bout, ln1g, ln1b, win, b_in, ln2g,
      ln2b, wg)


@jax.jit
def kernel(x, edge_index, ln1_g, ln1_b, Win, b_in, ln2_g, ln2_b, Wg, bg,
           Wout, bout):
    src = edge_index[0].astype(jnp.int32)
    dst = edge_index[1].astype(jnp.int32)
    npad = EPSP - EPS
    k = jnp.arange(npad, dtype=jnp.int32)
    padsrc = jnp.broadcast_to((k * 37) % N, (NS, npad))
    srcm = jnp.concatenate([src.reshape(NS, EPS), padsrc], axis=1)
    srcm = srcm.reshape(NS, 2, HALF, CH)
    padrows = (N + 15 * jnp.arange(NS, dtype=jnp.int32)[:, None]
               + (k[None, :] % 15))
    dstm = jnp.concatenate([dst.reshape(NS, EPS), padrows], axis=1)
    dstm = dstm.reshape(NS, 2, HALF, CH)
    zeros_deg = jnp.zeros((N,), jnp.float32)
    zeros_agg = jnp.zeros((ROWS_PT, H), jnp.float32)

    WinT = jnp.swapaxes(Win, 1, 2).astype(jnp.bfloat16)
    WgT = jnp.swapaxes(Wg, 1, 2).astype(jnp.bfloat16)
    WoutT = jnp.swapaxes(Wout, 1, 2).astype(jnp.bfloat16)

    deg2 = _deg(dst, zeros_deg).T

    h, gp = _stage_a(x, deg2, ln1_g[0:1], ln1_b[0:1], WinT[0], b_in[0:1],
                     ln2_g[0:1], ln2_b[0:1], WgT[0])
    for i in range(3):
        agg = _agg(gp, srcm, dstm, zeros_agg)
        if i < 2:
            x, h, gp = _stage_ba(x, h, gp, agg, deg2, bg[i:i + 1], WoutT[i],
                                 bout[i:i + 1], ln1_g[i + 1:i + 2],
                                 ln1_b[i + 1:i + 2], WinT[i + 1],
                                 b_in[i + 1:i + 2], ln2_g[i + 1:i + 2],
                                 ln2_b[i + 1:i + 2], WgT[i + 1])
        else:
            x = _stage_b(x, h, gp, agg, deg2, bg[i:i + 1], WoutT[i],
                         bout[i:i + 1])
    return x


# speedup vs baseline: 2.6333x; 1.0070x over previous
import functools
import math

import jax
import jax.numpy as jnp
from jax import lax
from jax.experimental import pallas as pl
from jax.experimental.pallas import tpu as pltpu
from jax.experimental.pallas import tpu_sc as plsc

N = 10000
E = 160000
D = 256
H = 128
NS = 16
NC = 2
CH = 126
EPS = E // NS
EPSP = 10080
NCHUNK = EPSP // CH
HALF = NCHUNK // 2
NP = 10240
DUMP = NP - 1
ROWS_PT = NP // NS
EPW = E // (NS * NC)
DEG_ROWS = N // 16
BLK = 1000
GRID = N // BLK

_mesh = plsc.VectorSubcoreMesh(core_axis_name="c", subcore_axis_name="s")


@functools.partial(
    pl.kernel,
    out_type=jax.ShapeDtypeStruct((NC * NS, N), jnp.float32),
    mesh=_mesh,
    scratch_types=[
        pltpu.VMEM((EPW + 16,), jnp.int32),
        pltpu.VMEM((N,), jnp.float32),
    ],
    compiler_params=pltpu.CompilerParams(needs_layout_passes=False),
)
def _deg(dst_hbm, zeros_hbm, out_hbm, dstv, priv):
    c = lax.axis_index("c")
    s = lax.axis_index("s")
    wid = s * NC + c
    pltpu.sync_copy(dst_hbm.at[pl.ds(wid * EPW, EPW)], dstv.at[pl.ds(0, EPW)])
    pltpu.sync_copy(zeros_hbm, priv)

    iota = lax.iota(jnp.int32, 16)
    ones = jnp.full((16,), 1.0, jnp.float32)

    def body(q, carry):
        dv = dstv[pl.ds(q * 16, 16)]
        msk = (iota + q * 16) < EPW
        plsc.addupdate_scatter(priv, [dv], ones, mask=msk)
        return carry

    lax.fori_loop(0, (EPW + 15) // 16, body, 0)
    pltpu.sync_copy(priv, out_hbm.at[wid])


@functools.partial(
    pl.kernel,
    out_type=jax.ShapeDtypeStruct((NC, NP, H), jnp.float32),
    mesh=_mesh,
    scratch_types=[
        pltpu.VMEM((HALF, CH), jnp.int32),
        pltpu.VMEM((HALF, CH), jnp.int32),
        pltpu.VMEM((CH, H), jnp.float32),
        pltpu.VMEM((CH, H), jnp.float32),
        pltpu.VMEM_SHARED((NP, H), jnp.float32),
        pltpu.SemaphoreType.DMA,
        pltpu.SemaphoreType.DMA,
    ],
    compiler_params=pltpu.CompilerParams(needs_layout_passes=False),
)
def _agg(gp_hbm, srcm_hbm, dstm_hbm, zeros_hbm, out_hbm,
         srcv, dstv, rows0, rows1, acc, sem0, sem1):
    c = lax.axis_index("c")
    s = lax.axis_index("s")

    gp = gp_hbm.at[c]

    def gather(r, buf, sem):
        pltpu.async_copy(gp.at[srcv.at[r]], buf, sem)

    def wait(r, buf, sem):
        pltpu.make_async_copy(gp.at[srcv.at[r]], buf, sem).wait()

    def scat(r, buf):
        pltpu.sync_copy(buf, acc.at[dstv.at[r]], add=True)

    for half in range(2):
        pltpu.sync_copy(srcm_hbm.at[s, half], srcv)
        pltpu.sync_copy(dstm_hbm.at[s, half], dstv)
        gather(0, rows0, sem0)
        if half == 0:
            pltpu.sync_copy(zeros_hbm, acc.at[pl.ds(s * ROWS_PT, ROWS_PT)])
            plsc.subcore_barrier()

        def body(g, carry):
            r = 2 * g
            gather(r + 1, rows1, sem1)
            wait(r, rows0, sem0)
            scat(r, rows0)
            gather(r + 2, rows0, sem0)
            wait(r + 1, rows1, sem1)
            scat(r + 1, rows1)
            return carry

        lax.fori_loop(0, HALF // 2 - 1, body, 0)
        gather(HALF - 1, rows1, sem1)
        wait(HALF - 2, rows0, sem0)
        scat(HALF - 2, rows0)
        wait(HALF - 1, rows1, sem1)
        scat(HALF - 1, rows1)
    plsc.subcore_barrier()
    pltpu.sync_copy(acc.at[pl.ds(s * ROWS_PT, ROWS_PT)],
                    out_hbm.at[c, pl.ds(s * ROWS_PT, ROWS_PT)])


def _ln(h, g, b):
    mu = jnp.mean(h, axis=-1, keepdims=True)
    var = jnp.mean((h - mu) ** 2, axis=-1, keepdims=True)
    return (h - mu) * lax.rsqrt(var + 1e-5) * g + b


_DNUM = (((1,), (0,)), ((), ()))


def _mm(a, w_ref):
    return lax.dot_general(a.astype(jnp.bfloat16), w_ref[...], _DNUM,
                           preferred_element_type=jnp.float32)


def _stage_a_body(x_ref, deg2_ref, ln1g_ref, ln1b_ref, win_ref, bin_ref,
                  ln2g_ref, ln2b_ref, wg_ref, h_out, gp_out):
    x = x_ref[...]
    deg = jnp.sum(deg2_ref[...], axis=1, keepdims=True) + 1.0
    dinv = lax.rsqrt(deg)
    h = _ln(x, ln1g_ref[...], ln1b_ref[...])
    h = _mm(h, win_ref) + bin_ref[...]
    h = 0.5 * h * (1.0 + lax.erf(h * (1.0 / math.sqrt(2.0))))
    g = _ln(h, ln2g_ref[...], ln2b_ref[...])
    g = _mm(g, wg_ref)
    gp = dinv * g
    h_out[...] = h
    gp_out[0] = gp[:, :H]
    gp_out[1] = gp[:, H:]


def _stage_a(x, deg2, ln1g, ln1b, win, b_in, ln2g, ln2b, wg):
    return pl.pallas_call(
        _stage_a_body,
        grid=(GRID,),
        in_specs=[
            pl.BlockSpec((BLK, D), lambda i: (i, 0)),
            pl.BlockSpec((BLK, NC * NS), lambda i: (i, 0)),
            pl.BlockSpec((1, D), lambda i: (0, 0)),
            pl.BlockSpec((1, D), lambda i: (0, 0)),
            pl.BlockSpec((D, D), lambda i: (0, 0)),
            pl.BlockSpec((1, D), lambda i: (0, 0)),
            pl.BlockSpec((1, D), lambda i: (0, 0)),
            pl.BlockSpec((1, D), lambda i: (0, 0)),
            pl.BlockSpec((D, D), lambda i: (0, 0)),
        ],
        out_specs=[
            pl.BlockSpec((BLK, D), lambda i: (i, 0)),
            pl.BlockSpec((NC, BLK, H), lambda i: (0, i, 0)),
        ],
        out_shape=[
            jax.ShapeDtypeStruct((N, D), jnp.float32),
            jax.ShapeDtypeStruct((NC, N, H), jnp.float32),
        ],
    )(x, deg2, ln1g, ln1b, win, b_in, ln2g, ln2b, wg)


def _stage_b_body(x_ref, h_ref, gp_ref, agg_ref, deg2_ref, bg_ref, wout_ref,
                  bout_ref, out_ref):
    deg = jnp.sum(deg2_ref[...], axis=1, keepdims=True) + 1.0
    dinv = lax.rsqrt(deg)
    aggf = jnp.concatenate(
        [agg_ref[0] + gp_ref[0], agg_ref[1] + gp_ref[1]], axis=1)
    gate = jnp.tanh(dinv * aggf + bg_ref[...])
    m = gate * h_ref[...]
    out_ref[...] = x_ref[...] + _mm(m, wout_ref) + bout_ref[...]


def _stage_b(x, h, gp, agg, deg2, bg, wout, bout):
    return pl.pallas_call(
        _stage_b_body,
        grid=(GRID,),
        in_specs=[
            pl.BlockSpec((BLK, D), lambda i: (i, 0)),
            pl.BlockSpec((BLK, D), lambda i: (i, 0)),
            pl.BlockSpec((NC, BLK, H), lambda i: (0, i, 0)),
            pl.BlockSpec((NC, BLK, H), lambda i: (0, i, 0)),
            pl.BlockSpec((BLK, NC * NS), lambda i: (i, 0)),
            pl.BlockSpec((1, D), lambda i: (0, 0)),
            pl.BlockSpec((D, D), lambda i: (0, 0)),
            pl.BlockSpec((1, D), lambda i: (0, 0)),
        ],
        out_specs=pl.BlockSpec((BLK, D), lambda i: (i, 0)),
        out_shape=jax.ShapeDtypeStruct((N, D), jnp.float32),
    )(x, h, gp, agg, deg2, bg, wout, bout)


def _stage_ba_body(x_ref, h_ref, gp_ref, agg_ref, deg2_ref, bg_ref, wout_ref,
                   bout_ref, ln1g_ref, ln1b_ref, win_ref, bin_ref, ln2g_ref,
                   ln2b_ref, wg_ref, x_out, h_out, gp_out):
    deg = jnp.sum(deg2_ref[...], axis=1, keepdims=True) + 1.0
    dinv = lax.rsqrt(deg)
    aggf = jnp.concatenate(
        [agg_ref[0] + gp_ref[0], agg_ref[1] + gp_ref[1]], axis=1)
    gate = jnp.tanh(dinv * aggf + bg_ref[...])
    m = gate * h_ref[...]
    xn = x_ref[...] + _mm(m, wout_ref) + bout_ref[...]
    x_out[...] = xn
    h = _ln(xn, ln1g_ref[...], ln1b_ref[...])
    h = _mm(h, win_ref) + bin_ref[...]
    h = 0.5 * h * (1.0 + lax.erf(h * (1.0 / math.sqrt(2.0))))
    g = _ln(h, ln2g_ref[...], ln2b_ref[...])
    g = _mm(g, wg_ref)
    gp = dinv * g
    h_out[...] = h
    gp_out[0] = gp[:, :H]
    gp_out[1] = gp[:, H:]


def _stage_ba(x, h, gp, agg, deg2, bg, wout, bout,
              ln1g, ln1b, win, b_in, ln2g, ln2b, wg):
    full = lambda i: (0, 0)
    return pl.pallas_call(
        _stage_ba_body,
        grid=(GRID,),
        in_specs=[
            pl.BlockSpec((BLK, D), lambda i: (i, 0)),
            pl.BlockSpec((BLK, D), lambda i: (i, 0)),
            pl.BlockSpec((NC, BLK, H), lambda i: (0, i, 0)),
            pl.BlockSpec((NC, BLK, H), lambda i: (0, i, 0)),
            pl.BlockSpec((BLK, NC * NS), lambda i: (i, 0)),
            pl.BlockSpec((1, D), full),
            pl.BlockSpec((D, D), full),
            pl.BlockSpec((1, D), full),
            pl.BlockSpec((1, D), full),
            pl.BlockSpec((1, D), full),
            pl.BlockSpec((D, D), full),
            pl.BlockSpec((1, D), full),
            pl.BlockSpec((1, D), full),
            pl.BlockSpec((1, D), full),
            pl.BlockSpec((D, D), full),
        ],
        out_specs=[
            pl.BlockSpec((BLK, D), lambda i: (i, 0)),
            pl.BlockSpec((BLK, D), lambda i: (i, 0)),
            pl.BlockSpec((NC, BLK, H), lambda i: (0, i, 0)),
        ],
        out_shape=[
            jax.ShapeDtypeStruct((N, D), jnp.float32),
            jax.ShapeDtypeStruct((N, D), jnp.float32),
            jax.ShapeDtypeStruct((NC, N, H), jnp.float32),
        ],
    )(x, h, gp, agg, deg2, bg, wout, bout, ln1g, ln1b, win, b_in, ln2g,
      ln2b, wg)


@jax.jit
def kernel(x, edge_index, ln1_g, ln1_b, Win, b_in, ln2_g, ln2_b, Wg, bg,
           Wout, bout):
    src = edge_index[0].astype(jnp.int32)
    dst = edge_index[1].astype(jnp.int32)
    npad = EPSP - EPS
    k = jnp.arange(npad, dtype=jnp.int32)
    padsrc = jnp.broadcast_to((k * 37) % N, (NS, npad))
    srcm = jnp.concatenate([src.reshape(NS, EPS), padsrc], axis=1)
    srcm = srcm.reshape(NS, 2, HALF, CH)
    padrows = (N + 15 * jnp.arange(NS, dtype=jnp.int32)[:, None]
               + (k[None, :] % 15))
    dstm = jnp.concatenate([dst.reshape(NS, EPS), padrows], axis=1)
    dstm = dstm.reshape(NS, 2, HALF, CH)
    zeros_deg = jnp.zeros((N,), jnp.float32)
    zeros_agg = jnp.zeros((ROWS_PT, H), jnp.float32)

    WinT = jnp.swapaxes(Win, 1, 2).astype(jnp.bfloat16)
    WgT = jnp.swapaxes(Wg, 1, 2).astype(jnp.bfloat16)
    WoutT = jnp.swapaxes(Wout, 1, 2).astype(jnp.bfloat16)

    deg2 = _deg(dst, zeros_deg).T

    h, gp = _stage_a(x, deg2, ln1_g[0:1], ln1_b[0:1], WinT[0], b_in[0:1],
                     ln2_g[0:1], ln2_b[0:1], WgT[0])
    for i in range(3):
        agg = _agg(gp, srcm, dstm, zeros_agg)
        if i < 2:
            x, h, gp = _stage_ba(x, h, gp, agg, deg2, bg[i:i + 1], WoutT[i],
                                 bout[i:i + 1], ln1_g[i + 1:i + 2],
                                 ln1_b[i + 1:i + 2], WinT[i + 1],
                                 b_in[i + 1:i + 2], ln2_g[i + 1:i + 2],
                                 ln2_b[i + 1:i + 2], WgT[i + 1])
        else:
            x = _stage_b(x, h, gp, agg, deg2, bg[i:i + 1], WoutT[i],
                         bout[i:i + 1])
    return x
